# Initial kernel scaffold; baseline (speedup 1.0000x reference)
import jax
import jax.numpy as jnp
from jax.experimental import pallas as pl


def kernel(tokens, positions, token_table, position_table):
    raise NotImplementedError("write your pallas kernel here")


# speedup vs baseline: 1.1562x; 1.1562x over previous
import functools

import jax
import jax.numpy as jnp
from jax import lax
from jax.experimental import pallas as pl
from jax.experimental.pallas import tpu as pltpu
from jax.experimental.pallas import tpu_sc as plsc

VOCAB = 100000
EMBED = 128
SEQ_LEN = 2048
BATCH = 4

B = BATCH * SEQ_LEN
NC = 2
NS = 16
NW = NC * NS
BPW = B // NW
LANES = 16


def _emb_body(tok_hbm, pos_hbm, ttab_hbm, ptab_hbm, out_hbm,
              tok_v, pos_v, trows, prows, sem_t, sem_p):
    wid = lax.axis_index("s") * NC + lax.axis_index("c")
    base = wid * BPW

    pltpu.sync_copy(tok_hbm.at[pl.ds(base, BPW)], tok_v)
    pltpu.sync_copy(pos_hbm.at[pl.ds(base, BPW)], pos_v)

    cp_t = pltpu.async_copy(ttab_hbm.at[tok_v], trows, sem_t)
    cp_p = pltpu.async_copy(ptab_hbm.at[pos_v], prows, sem_p)
    cp_t.wait()
    cp_p.wait()

    def add_row(r, carry):
        for c in range(EMBED // LANES):
            sl = pl.ds(c * LANES, LANES)
            trows[r, sl] = trows[r, sl] + prows[r, sl]
        return carry

    lax.fori_loop(0, BPW, add_row, 0, unroll=2)

    pltpu.sync_copy(trows, out_hbm.at[pl.ds(base, BPW)])


@jax.jit
def _emb_call(tok_flat, pos_flat, token_table, position_table):
    mesh = plsc.VectorSubcoreMesh(core_axis_name="c", subcore_axis_name="s")
    kfn = functools.partial(
        pl.kernel,
        mesh=mesh,
        out_type=jax.ShapeDtypeStruct((B, EMBED), jnp.float32),
        scratch_types=[
            pltpu.VMEM((BPW,), jnp.int32),
            pltpu.VMEM((BPW,), jnp.int32),
            pltpu.VMEM((BPW, EMBED), jnp.float32),
            pltpu.VMEM((BPW, EMBED), jnp.float32),
            pltpu.SemaphoreType.DMA,
            pltpu.SemaphoreType.DMA,
        ],
    )(_emb_body)
    return kfn(tok_flat, pos_flat, token_table, position_table)


def kernel(tokens, positions, token_table, position_table):
    tok_flat = jnp.reshape(tokens, (B,)).astype(jnp.int32)
    pos_flat = jnp.reshape(positions, (B,)).astype(jnp.int32)
    out = _emb_call(tok_flat, pos_flat, token_table, position_table)
    return jnp.reshape(out, (BATCH, SEQ_LEN, EMBED))


# speedup vs baseline: 1.4624x; 1.2648x over previous
import functools

import jax
import jax.numpy as jnp
from jax import lax
from jax.experimental import pallas as pl
from jax.experimental.pallas import tpu as pltpu
from jax.experimental.pallas import tpu_sc as plsc

VOCAB = 100000
EMBED = 128
SEQ_LEN = 2048
BATCH = 4

B = BATCH * SEQ_LEN
NC = 2
NS = 16
NW = NC * NS
BPW = B // NW
LANES = 16
NCHUNK = 4
CR = BPW // NCHUNK


def _emb_body(tok_hbm, pos_hbm, ttab_hbm, ptab_hbm, out_hbm,
              tok_v, pos_v, trows, prows,
              sem_t0, sem_t1, sem_p0, sem_p1, sem_o):
    wid = lax.axis_index("s") * NC + lax.axis_index("c")
    base = wid * BPW

    sems_t = (sem_t0, sem_t1)
    sems_p = (sem_p0, sem_p1)

    pltpu.sync_copy(tok_hbm.at[pl.ds(base, BPW)], tok_v)
    pltpu.sync_copy(pos_hbm.at[pl.ds(base, BPW)], pos_v)

    def gather_chunk(c):
        rs = pl.ds(c * CR, CR)
        pltpu.async_copy(ttab_hbm.at[tok_v.at[rs]], trows.at[rs], sems_t[c % 2])
        pltpu.async_copy(ptab_hbm.at[pos_v.at[rs]], prows.at[rs], sems_p[c % 2])

    def wait_chunk(c):
        rs = pl.ds(c * CR, CR)
        pltpu.make_async_copy(ttab_hbm.at[tok_v.at[rs]], trows.at[rs], sems_t[c % 2]).wait()
        pltpu.make_async_copy(ptab_hbm.at[pos_v.at[rs]], prows.at[rs], sems_p[c % 2]).wait()

    gather_chunk(0)
    for c in range(NCHUNK):
        if c + 1 < NCHUNK:
            gather_chunk(c + 1)
        wait_chunk(c)

        @plsc.parallel_loop(c * CR, (c + 1) * CR, step=1, unroll=2)
        def add_row(r):
            for k in range(EMBED // LANES):
                sl = pl.ds(k * LANES, LANES)
                plsc.addupdate(trows.at[r, sl], prows[r, sl])

        rs = pl.ds(c * CR, CR)
        pltpu.async_copy(trows.at[rs], out_hbm.at[pl.ds(base + c * CR, CR)], sem_o)

    for c in range(NCHUNK):
        rs = pl.ds(c * CR, CR)
        pltpu.make_async_copy(trows.at[rs], out_hbm.at[pl.ds(base + c * CR, CR)], sem_o).wait()


@jax.jit
def _emb_call(tok_flat, pos_flat, token_table, position_table):
    mesh = plsc.VectorSubcoreMesh(core_axis_name="c", subcore_axis_name="s")
    kfn = functools.partial(
        pl.kernel,
        mesh=mesh,
        out_type=jax.ShapeDtypeStruct((B, EMBED), jnp.float32),
        scratch_types=[
            pltpu.VMEM((BPW,), jnp.int32),
            pltpu.VMEM((BPW,), jnp.int32),
            pltpu.VMEM((BPW, EMBED), jnp.float32),
            pltpu.VMEM((BPW, EMBED), jnp.float32),
            pltpu.SemaphoreType.DMA,
            pltpu.SemaphoreType.DMA,
            pltpu.SemaphoreType.DMA,
            pltpu.SemaphoreType.DMA,
            pltpu.SemaphoreType.DMA,
        ],
    )(_emb_body)
    return kfn(tok_flat, pos_flat, token_table, position_table)


def kernel(tokens, positions, token_table, position_table):
    tok_flat = jnp.reshape(tokens, (B,)).astype(jnp.int32)
    pos_flat = jnp.reshape(positions, (B,)).astype(jnp.int32)
    out = _emb_call(tok_flat, pos_flat, token_table, position_table)
    return jnp.reshape(out, (BATCH, SEQ_LEN, EMBED))
